# baseline (speedup 1.0000x reference)
import functools

import jax
import jax.numpy as jnp
from jax import lax
from jax.experimental import pallas as pl
from jax.experimental.pallas import tpu as pltpu
from jax.experimental.pallas import tpu_sc as plsc

_NR = 1000
_D = 128
_B = 16384
_NC = 2
_NS = 16
_NW = _NC * _NS
_BPW = _B // _NW
_C = 128
_NCHUNK = _BPW // _C
_L = 16


def _sqrt16(x):
    i = lax.bitcast_convert_type(x, jnp.int32)
    y = lax.bitcast_convert_type(0x5F3759DF - (i >> 1), jnp.float32)
    for _ in range(3):
        y = y * (1.5 - 0.5 * x * y * y)
    return x * y


def _tec_body(ent, rel, heads, rels, tails, out,
              idx_h, idx_r, idx_t, bh, br, bt, out_v, sems):
    wid = lax.axis_index("s") * _NC + lax.axis_index("c")
    base = wid * _BPW

    pltpu.sync_copy(heads.at[pl.ds(base, _BPW)], idx_h)
    pltpu.sync_copy(rels.at[pl.ds(base, _BPW)], idx_r)
    pltpu.sync_copy(tails.at[pl.ds(base, _BPW)], idx_t)

    def fire(c, slot):
        off = c * _C
        pltpu.async_copy(ent.at[idx_h.at[pl.ds(off, _C)]], bh.at[slot],
                         sems.at[slot])
        pltpu.async_copy(rel.at[idx_r.at[pl.ds(off, _C)]], br.at[slot],
                         sems.at[slot])
        pltpu.async_copy(ent.at[idx_t.at[pl.ds(off, _C)]], bt.at[slot],
                         sems.at[slot])

    def drain(c, slot):
        off = c * _C
        pltpu.make_async_copy(ent.at[idx_h.at[pl.ds(off, _C)]], bh.at[slot],
                              sems.at[slot]).wait()
        pltpu.make_async_copy(rel.at[idx_r.at[pl.ds(off, _C)]], br.at[slot],
                              sems.at[slot]).wait()
        pltpu.make_async_copy(ent.at[idx_t.at[pl.ds(off, _C)]], bt.at[slot],
                              sems.at[slot]).wait()

    lane = lax.iota(jnp.int32, _L)
    fire(0, 0)

    def chunk_body(c, carry):
        slot = lax.rem(c, 2)

        @pl.when(c + 1 < _NCHUNK)
        def _():
            fire(c + 1, 1 - slot)

        drain(c, slot)

        def group_body(g, carry2):
            def row_ins(ii, vec):
                i = g * _L + ii
                acc0 = jnp.zeros((_L,), jnp.float32)
                acc1 = jnp.zeros((_L,), jnp.float32)
                for j in range(0, _D // _L, 2):
                    sl0 = pl.ds(j * _L, _L)
                    sl1 = pl.ds((j + 1) * _L, _L)
                    s0 = (bh[slot, i, sl0] + br[slot, i, sl0]) \
                        - bt[slot, i, sl0]
                    s1 = (bh[slot, i, sl1] + br[slot, i, sl1]) \
                        - bt[slot, i, sl1]
                    acc0 = acc0 + s0 * s0
                    acc1 = acc1 + s1 * s1
                acc = acc0 + acc1
                for sh in (8, 4, 2, 1):
                    acc = acc + acc.at[lane ^ sh].get(
                        mode="promise_in_bounds")
                return jnp.where(lane == ii, acc, vec)

            vec = lax.fori_loop(0, _L, row_ins,
                                jnp.zeros((_L,), jnp.float32), unroll=2)
            out_v[pl.ds(c * _C + g * _L, _L)] = _sqrt16(vec)
            return carry2

        lax.fori_loop(0, _C // _L, group_body, 0)
        return carry

    lax.fori_loop(0, _NCHUNK, chunk_body, 0)

    pltpu.sync_copy(out_v, out.at[pl.ds(base, _BPW)])


_kg_call = functools.partial(
    pl.kernel,
    mesh=plsc.VectorSubcoreMesh(core_axis_name="c", subcore_axis_name="s"),
    out_type=jax.ShapeDtypeStruct((_B,), jnp.float32),
    scratch_types=[
        pltpu.VMEM((_BPW,), jnp.int32),
        pltpu.VMEM((_BPW,), jnp.int32),
        pltpu.VMEM((_BPW,), jnp.int32),
        pltpu.VMEM((2, _C, _D), jnp.float32),
        pltpu.VMEM((2, _C, _D), jnp.float32),
        pltpu.VMEM((2, _C, _D), jnp.float32),
        pltpu.VMEM((_BPW,), jnp.float32),
        pltpu.SemaphoreType.DMA((2,)),
    ],
)(_tec_body)


def kernel(entity_emb, relation_emb, heads, relations, tails):
    h = heads.astype(jnp.int32)
    r = relations.astype(jnp.int32)
    t = tails.astype(jnp.int32)
    return _kg_call(entity_emb, relation_emb, h, r, t)


# speedup vs baseline: 1.0225x; 1.0225x over previous
import functools

import jax
import jax.numpy as jnp
from jax import lax
from jax.experimental import pallas as pl
from jax.experimental.pallas import tpu as pltpu
from jax.experimental.pallas import tpu_sc as plsc

_NR = 1000
_D = 128
_B = 16384
_NC = 2
_NS = 16
_NW = _NC * _NS
_BPW = _B // _NW
_C = 128
_NCHUNK = _BPW // _C
_L = 16


def _sqrt16(x):
    i = lax.bitcast_convert_type(x, jnp.int32)
    y = lax.bitcast_convert_type(0x5F3759DF - (i >> 1), jnp.float32)
    for _ in range(3):
        y = y * (1.5 - 0.5 * x * y * y)
    return x * y


def _tec_body(ent, rel, heads, rels, tails, out,
              idx_h, idx_r, idx_t, bh, br, bt, out_v, sems):
    wid = lax.axis_index("s") * _NC + lax.axis_index("c")
    base = wid * _BPW

    pltpu.sync_copy(heads.at[pl.ds(base, _BPW)], idx_h)
    pltpu.sync_copy(rels.at[pl.ds(base, _BPW)], idx_r)
    pltpu.sync_copy(tails.at[pl.ds(base, _BPW)], idx_t)

    def fire(c, slot):
        off = c * _C
        pltpu.async_copy(ent.at[idx_h.at[pl.ds(off, _C)]], bh.at[slot],
                         sems.at[slot])
        pltpu.async_copy(rel.at[idx_r.at[pl.ds(off, _C)]], br.at[slot],
                         sems.at[slot])
        pltpu.async_copy(ent.at[idx_t.at[pl.ds(off, _C)]], bt.at[slot],
                         sems.at[slot])

    def drain(c, slot):
        off = c * _C
        pltpu.make_async_copy(ent.at[idx_h.at[pl.ds(off, _C)]], bh.at[slot],
                              sems.at[slot]).wait()
        pltpu.make_async_copy(rel.at[idx_r.at[pl.ds(off, _C)]], br.at[slot],
                              sems.at[slot]).wait()
        pltpu.make_async_copy(ent.at[idx_t.at[pl.ds(off, _C)]], bt.at[slot],
                              sems.at[slot]).wait()

    lane = lax.iota(jnp.int32, _L)
    fire(0, 0)

    def chunk_body(c, carry):
        slot = lax.rem(c, 2)
        drain(c, slot)

        @pl.when(c + 1 < _NCHUNK)
        def _():
            fire(c + 1, 1 - slot)

        def group_body(g, carry2):
            def row_ins(ii, vec):
                i = g * _L + ii
                acc = jnp.zeros((_L,), jnp.float32)
                for j in range(_D // _L):
                    sl = pl.ds(j * _L, _L)
                    s = (bh[slot, i, sl] + br[slot, i, sl]) \
                        - bt[slot, i, sl]
                    acc = acc + s * s
                for sh in (8, 4, 2, 1):
                    acc = acc + acc.at[lane ^ sh].get(
                        mode="promise_in_bounds")
                return jnp.where(lane == ii, acc, vec)

            vec = lax.fori_loop(0, _L, row_ins,
                                jnp.zeros((_L,), jnp.float32), unroll=1)
            out_v[pl.ds(c * _C + g * _L, _L)] = _sqrt16(vec)
            return carry2

        lax.fori_loop(0, _C // _L, group_body, 0)
        return carry

    lax.fori_loop(0, _NCHUNK, chunk_body, 0)

    pltpu.sync_copy(out_v, out.at[pl.ds(base, _BPW)])


_kg_call = functools.partial(
    pl.kernel,
    mesh=plsc.VectorSubcoreMesh(core_axis_name="c", subcore_axis_name="s"),
    out_type=jax.ShapeDtypeStruct((_B,), jnp.float32),
    scratch_types=[
        pltpu.VMEM((_BPW,), jnp.int32),
        pltpu.VMEM((_BPW,), jnp.int32),
        pltpu.VMEM((_BPW,), jnp.int32),
        pltpu.VMEM((2, _C, _D), jnp.float32),
        pltpu.VMEM((2, _C, _D), jnp.float32),
        pltpu.VMEM((2, _C, _D), jnp.float32),
        pltpu.VMEM((_BPW,), jnp.float32),
        pltpu.SemaphoreType.DMA((2,)),
    ],
)(_tec_body)


def kernel(entity_emb, relation_emb, heads, relations, tails):
    h = heads.astype(jnp.int32)
    r = relations.astype(jnp.int32)
    t = tails.astype(jnp.int32)
    return _kg_call(entity_emb, relation_emb, h, r, t)


# speedup vs baseline: 1.0527x; 1.0295x over previous
import functools

import jax
import jax.numpy as jnp
from jax import lax
from jax.experimental import pallas as pl
from jax.experimental.pallas import tpu as pltpu
from jax.experimental.pallas import tpu_sc as plsc

_NR = 1000
_D = 128
_B = 16384
_NC = 2
_NS = 16
_NW = _NC * _NS
_BPW = _B // _NW
_C = 128
_NCHUNK = _BPW // _C
_L = 16


def _sqrt16(x):
    i = lax.bitcast_convert_type(x, jnp.int32)
    y = lax.bitcast_convert_type(0x5F3759DF - (i >> 1), jnp.float32)
    for _ in range(3):
        y = y * (1.5 - 0.5 * x * y * y)
    return x * y


def _tec_body(ent, rel, heads, rels, tails, out,
              idx_h, idx_r, idx_t, bh, br, bt, out_v, sems):
    wid = lax.axis_index("s") * _NC + lax.axis_index("c")
    base = wid * _BPW

    cph = pltpu.async_copy(heads.at[pl.ds(base, _BPW)], idx_h, sems.at[0])
    cpr = pltpu.async_copy(rels.at[pl.ds(base, _BPW)], idx_r, sems.at[1])
    cpt = pltpu.async_copy(tails.at[pl.ds(base, _BPW)], idx_t, sems.at[0])
    cph.wait()
    cpr.wait()
    cpt.wait()

    def fire(c, slot):
        off = c * _C
        pltpu.async_copy(ent.at[idx_h.at[pl.ds(off, _C)]], bh.at[slot],
                         sems.at[slot])
        pltpu.async_copy(rel.at[idx_r.at[pl.ds(off, _C)]], br.at[slot],
                         sems.at[slot])
        pltpu.async_copy(ent.at[idx_t.at[pl.ds(off, _C)]], bt.at[slot],
                         sems.at[slot])

    def drain(c, slot):
        off = c * _C
        pltpu.make_async_copy(ent.at[idx_h.at[pl.ds(off, _C)]], bh.at[slot],
                              sems.at[slot]).wait()
        pltpu.make_async_copy(rel.at[idx_r.at[pl.ds(off, _C)]], br.at[slot],
                              sems.at[slot]).wait()
        pltpu.make_async_copy(ent.at[idx_t.at[pl.ds(off, _C)]], bt.at[slot],
                              sems.at[slot]).wait()

    lane = lax.iota(jnp.int32, _L)
    fire(0, 0)

    def chunk_body(c, carry):
        slot = lax.rem(c, 2)
        drain(c, slot)

        @pl.when(c + 1 < _NCHUNK)
        def _():
            fire(c + 1, 1 - slot)

        def group_body(g, carry2):
            def row_ins(ii, vec):
                i = g * _L + ii
                acc = jnp.zeros((_L,), jnp.float32)
                for j in range(_D // _L):
                    sl = pl.ds(j * _L, _L)
                    s = (bh[slot, i, sl] + br[slot, i, sl]) \
                        - bt[slot, i, sl]
                    acc = acc + s * s
                for sh in (8, 4, 2, 1):
                    acc = acc + acc.at[lane ^ sh].get(
                        mode="promise_in_bounds")
                return jnp.where(lane == ii, acc, vec)

            vec = lax.fori_loop(0, _L, row_ins,
                                jnp.zeros((_L,), jnp.float32), unroll=1)
            out_v[pl.ds(c * _C + g * _L, _L)] = _sqrt16(vec)
            return carry2

        lax.fori_loop(0, _C // _L, group_body, 0)
        return carry

    lax.fori_loop(0, _NCHUNK, chunk_body, 0)

    pltpu.sync_copy(out_v, out.at[pl.ds(base, _BPW)])


_kg_call = functools.partial(
    pl.kernel,
    mesh=plsc.VectorSubcoreMesh(core_axis_name="c", subcore_axis_name="s"),
    out_type=jax.ShapeDtypeStruct((_B,), jnp.float32),
    scratch_types=[
        pltpu.VMEM((_BPW,), jnp.int32),
        pltpu.VMEM((_BPW,), jnp.int32),
        pltpu.VMEM((_BPW,), jnp.int32),
        pltpu.VMEM((2, _C, _D), jnp.float32),
        pltpu.VMEM((2, _C, _D), jnp.float32),
        pltpu.VMEM((2, _C, _D), jnp.float32),
        pltpu.VMEM((_BPW,), jnp.float32),
        pltpu.SemaphoreType.DMA((2,)),
    ],
)(_tec_body)


def kernel(entity_emb, relation_emb, heads, relations, tails):
    h = heads.astype(jnp.int32)
    r = relations.astype(jnp.int32)
    t = tails.astype(jnp.int32)
    return _kg_call(entity_emb, relation_emb, h, r, t)
